# baseline (speedup 1.0000x reference)
import functools

import jax
import jax.numpy as jnp
from jax import lax
from jax.experimental import pallas as pl
from jax.experimental.pallas import tpu as pltpu
from jax.experimental.pallas import tpu_sc as plsc

_C = 80
_CP = 128
_BR = 256


def _gather_rows_sc(table, indexes):
    _, d = table.shape
    b = indexes.shape[0]
    info = plsc.get_sparse_core_info()
    nw = info.num_cores * info.num_subcores
    b_per_w = b // nw
    mesh = plsc.VectorSubcoreMesh(core_axis_name="c", subcore_axis_name="s")

    @functools.partial(
        pl.kernel,
        mesh=mesh,
        out_type=jax.ShapeDtypeStruct((b, d), jnp.float32),
        scratch_types=[
            pltpu.VMEM((b_per_w,), jnp.int32),
            pltpu.VMEM((b_per_w, d), jnp.float32),
            pltpu.SemaphoreType.DMA,
        ],
    )
    def gather_kernel(table_hbm, idx_hbm, out_hbm, idx_v, rows_v, sem):
        wid = lax.axis_index("s") * info.num_cores + lax.axis_index("c")
        base = wid * b_per_w
        pltpu.sync_copy(idx_hbm.at[pl.ds(base, b_per_w)], idx_v)
        pltpu.async_copy(table_hbm.at[idx_v], rows_v, sem).wait()
        pltpu.sync_copy(rows_v, out_hbm.at[pl.ds(base, b_per_w)])

    return gather_kernel(table, indexes)


def _sweep_tc(embed_sim, idx2d, labels):
    b, n = embed_sim.shape
    nr = b // _BR

    def body(x_ref, idx_ref, lab_ref, m_ref, z_ref):
        idx = idx_ref[...]
        cols = lax.broadcasted_iota(jnp.int32, (_BR, n), 1)
        e = jnp.exp(x_ref[...])
        e = jnp.where(cols == idx, 0.0, e)
        m_ref[...] = jnp.dot(e, lab_ref[...], preferred_element_type=jnp.float32)
        z_ref[...] = jnp.sum(e, axis=1, keepdims=True)

    return pl.pallas_call(
        body,
        grid=(nr,),
        in_specs=[
            pl.BlockSpec((_BR, n), lambda i: (i, 0)),
            pl.BlockSpec((_BR, 1), lambda i: (i, 0)),
            pl.BlockSpec((n, _C), lambda i: (0, 0)),
        ],
        out_specs=[
            pl.BlockSpec((_BR, _C), lambda i: (i, 0)),
            pl.BlockSpec((_BR, 1), lambda i: (i, 0)),
        ],
        out_shape=[
            jax.ShapeDtypeStruct((b, _C), jnp.float32),
            jax.ShapeDtypeStruct((b, 1), jnp.float32),
        ],
        compiler_params=pltpu.CompilerParams(
            dimension_semantics=("arbitrary",),
        ),
    )(embed_sim, idx2d, labels)


def _epilogue_tc(m, z, gathered):
    b = m.shape[0]
    inv_b = -1.0 / b
    inv_c = 1.0 / _C

    def body(m_ref, z_ref, g_ref, out_ref):
        p = jnp.sum(m_ref[...] * g_ref[:, pl.ds(0, _C)], axis=1,
                    keepdims=True) * inv_c
        prob = p / z_ref[...]
        ll = jnp.log(jnp.where(prob != 0.0, prob, 1.0))
        out_ref[0, 0] = jnp.sum(ll) * inv_b

    return pl.pallas_call(
        body,
        out_specs=pl.BlockSpec(memory_space=pltpu.SMEM),
        out_shape=jax.ShapeDtypeStruct((1, 1), jnp.float32),
    )(m, z, gathered)


def kernel(embed_sim, indexes, labels):
    b, _ = embed_sim.shape
    table = jnp.pad(labels, ((0, 0), (0, _CP - _C)))
    g = _gather_rows_sc(table, indexes)
    m, z = _sweep_tc(embed_sim, indexes.reshape(b, 1), labels)
    out = _epilogue_tc(m, z, g)
    return out[0, 0]


# speedup vs baseline: 1.0400x; 1.0400x over previous
import functools

import jax
import jax.numpy as jnp
from jax import lax
from jax.experimental import pallas as pl
from jax.experimental.pallas import tpu as pltpu
from jax.experimental.pallas import tpu_sc as plsc

_C = 80
_CP = 128
_BR = 512


def _gather_rows_sc(table, indexes):
    _, d = table.shape
    b = indexes.shape[0]
    info = plsc.get_sparse_core_info()
    nw = info.num_cores * info.num_subcores
    b_per_w = b // nw
    mesh = plsc.VectorSubcoreMesh(core_axis_name="c", subcore_axis_name="s")

    @functools.partial(
        pl.kernel,
        mesh=mesh,
        out_type=jax.ShapeDtypeStruct((b, d), jnp.float32),
        scratch_types=[
            pltpu.VMEM((b_per_w,), jnp.int32),
            pltpu.VMEM((b_per_w, d), jnp.float32),
            pltpu.SemaphoreType.DMA,
        ],
    )
    def gather_kernel(table_hbm, idx_hbm, out_hbm, idx_v, rows_v, sem):
        wid = lax.axis_index("s") * info.num_cores + lax.axis_index("c")
        base = wid * b_per_w
        pltpu.sync_copy(idx_hbm.at[pl.ds(base, b_per_w)], idx_v)
        pltpu.async_copy(table_hbm.at[idx_v], rows_v, sem).wait()
        pltpu.sync_copy(rows_v, out_hbm.at[pl.ds(base, b_per_w)])

    return gather_kernel(table, indexes)


def _sweep_tc(embed_sim, idx2d, labels):
    b, n = embed_sim.shape
    nr = b // _BR

    def body(x_ref, idx_ref, lab_ref, m_ref, z_ref):
        idx = idx_ref[...]
        cols = lax.broadcasted_iota(jnp.int32, (_BR, n), 1)
        e = jnp.exp(x_ref[...])
        e = jnp.where(cols == idx, 0.0, e)
        m_ref[...] = jnp.dot(e, lab_ref[...], preferred_element_type=jnp.float32)
        z_ref[...] = jnp.sum(e, axis=1, keepdims=True)

    return pl.pallas_call(
        body,
        grid=(nr,),
        in_specs=[
            pl.BlockSpec((_BR, n), lambda i: (i, 0)),
            pl.BlockSpec((_BR, 1), lambda i: (i, 0)),
            pl.BlockSpec((n, _C), lambda i: (0, 0)),
        ],
        out_specs=[
            pl.BlockSpec((_BR, _C), lambda i: (i, 0)),
            pl.BlockSpec((_BR, 1), lambda i: (i, 0)),
        ],
        out_shape=[
            jax.ShapeDtypeStruct((b, _C), jnp.float32),
            jax.ShapeDtypeStruct((b, 1), jnp.float32),
        ],
        compiler_params=pltpu.CompilerParams(
            dimension_semantics=("arbitrary",),
        ),
    )(embed_sim, idx2d, labels)


def _epilogue_tc(m, z, gathered):
    b = m.shape[0]
    inv_b = -1.0 / b
    inv_c = 1.0 / _C

    def body(m_ref, z_ref, g_ref, out_ref):
        p = jnp.sum(m_ref[...] * g_ref[:, pl.ds(0, _C)], axis=1,
                    keepdims=True) * inv_c
        prob = p / z_ref[...]
        ll = jnp.log(jnp.where(prob != 0.0, prob, 1.0))
        out_ref[0, 0] = jnp.sum(ll) * inv_b

    return pl.pallas_call(
        body,
        out_specs=pl.BlockSpec(memory_space=pltpu.SMEM),
        out_shape=jax.ShapeDtypeStruct((1, 1), jnp.float32),
    )(m, z, gathered)


def kernel(embed_sim, indexes, labels):
    b, _ = embed_sim.shape
    table = jnp.pad(labels, ((0, 0), (0, _CP - _C)))
    g = _gather_rows_sc(table, indexes)
    m, z = _sweep_tc(embed_sim, indexes.reshape(b, 1), labels)
    out = _epilogue_tc(m, z, g)
    return out[0, 0]
